# baseline (speedup 1.0000x reference)
import jax
import jax.numpy as jnp
from jax import lax
from jax.experimental import pallas as pl
from jax.experimental.pallas import tpu as pltpu
from jax.experimental.pallas import tpu_sc as plsc

H = 16
Q = 2048
K = 2048
T = 257
D = (T - 1) // 2
TPAD = 264
GPAD = 4128
R = 16
W = 4096
ROWS_PER_TILE = Q // 2
NBLK = ROWS_PER_TILE // R

M = K - 1
FILL_HI = 1904
FILL_LO = 2176


def _bcast_lane(v, lane):
    idx = jnp.full((16, 1), lane, jnp.int32)
    dnums = lax.GatherDimensionNumbers(
        offset_dims=(), collapsed_slice_dims=(0,), start_index_map=(0,)
    )
    return lax.gather(v, idx, dnums, slice_sizes=(1,),
                      mode=lax.GatherScatterMode.PROMISE_IN_BOUNDS)


NSHIFT = 8


def _rpb_sc(bias_hbm, out_hbm, col_v, src_sh, *gs_and_sem):
    gs = gs_and_sem[:NSHIFT]
    sem = gs_and_sem[NSHIFT]
    c = lax.axis_index("c")
    s = lax.axis_index("s")
    src_v = src_sh.at[s]
    pltpu.sync_copy(bias_hbm.at[s], col_v)

    c_hi = _bcast_lane(col_v[pl.ds(248, 16)], 8)
    c_lo = _bcast_lane(col_v[pl.ds(0, 16)], 0)

    for a in range(NSHIFT):
        g_a = gs[a]

        def fill_hi(u, carry, g_a=g_a):
            g_a[pl.ds(u * 16, 16)] = c_hi
            return carry

        def fill_lo(u, carry, g_a=g_a):
            g_a[pl.ds(FILL_LO + u * 16, 16)] = c_lo
            return carry

        lax.fori_loop(0, FILL_HI // 16, fill_hi, 0)
        lax.fori_loop(0, (GPAD - FILL_LO) // 16, fill_lo, 0)

    g0 = gs[0]
    g0[pl.ds(FILL_HI, 16)] = c_hi
    for w0 in range(1920, FILL_LO, 16):
        g0[pl.ds(w0, 16)] = lax.rev(col_v[pl.ds(2160 - w0, 16)], (0,))

    for a in range(1, NSHIFT):
        g_a = gs[a]
        for w0 in range(FILL_HI, FILL_LO, 16):
            g_a[pl.ds(w0, 16)] = g0[pl.ds(w0 + a, 16)]

    for r in range(R):
        shift = R - 1 - r
        a, q8 = shift % NSHIFT, (shift // NSHIFT) * NSHIFT
        pltpu.sync_copy(gs[a].at[pl.ds(q8, W)], src_v.at[r])

    def blk(b, carry):
        descs = []
        for t in range(4):
            i0 = c * ROWS_PER_TILE + (b * 4 + t) * R
            p = pl.multiple_of(M - (R - 1) - i0, 16)
            row0 = pl.multiple_of(s * Q + i0, 16)
            descs.append(
                pltpu.async_copy(
                    src_v.at[:, pl.ds(p, K)], out_hbm.at[pl.ds(row0, R), :], sem
                )
            )
        for dcp in descs:
            dcp.wait()
        return carry

    lax.fori_loop(0, NBLK // 4, blk, 0)


@jax.jit
def _launch(bias):
    bias_t = jnp.pad(bias.T, ((0, 0), (0, TPAD - T)))
    fn = pl.kernel(
        _rpb_sc,
        mesh=plsc.VectorSubcoreMesh(core_axis_name="c", subcore_axis_name="s"),
        out_type=jax.ShapeDtypeStruct((H * Q, K), jnp.float32),
        scratch_types=[
            pltpu.VMEM((TPAD,), jnp.float32),
            pltpu.MemorySpace.VMEM_SHARED((H, R, W), jnp.float32),
        ]
        + [pltpu.VMEM((GPAD,), jnp.float32) for _ in range(NSHIFT)]
        + [pltpu.SemaphoreType.DMA],
        compiler_params=pltpu.CompilerParams(use_tc_tiling_on_sc=False),
    )
    return fn(bias_t).reshape(H, Q, K)


def kernel(q_len, k_len, bias):
    return _launch(bias)


# speedup vs baseline: 3.5821x; 3.5821x over previous
import jax
import jax.numpy as jnp
from jax import lax
from jax.experimental import pallas as pl
from jax.experimental.pallas import tpu as pltpu
from jax.experimental.pallas import tpu_sc as plsc

H = 16
Q = 2048
K = 2048
T = 257
D = (T - 1) // 2
TPAD = 264
GPAD = 4128
M = K - 1
FILL_HI = 1904
FILL_LO = 2176
NKT = K // 128
CRUN = 8
NBLK = Q // 128


def _bcast_lane(v, lane):
    idx = jnp.full((16, 1), lane, jnp.int32)
    dnums = lax.GatherDimensionNumbers(
        offset_dims=(), collapsed_slice_dims=(0,), start_index_map=(0,)
    )
    return lax.gather(v, idx, dnums, slice_sizes=(1,),
                      mode=lax.GatherScatterMode.PROMISE_IN_BOUNDS)


def _rpb_sc(bias_hbm, out_hbm, col_v, g_v, bsrc_v, chi_v, clo_v, sem):
    c = lax.axis_index("c")
    s = lax.axis_index("s")
    pltpu.sync_copy(bias_hbm.at[s], col_v)

    c_hi = _bcast_lane(col_v[pl.ds(248, 16)], 8)
    c_lo = _bcast_lane(col_v[pl.ds(0, 16)], 0)

    def fill_g_hi(u, carry):
        g_v[pl.ds(u * 16, 16)] = c_hi
        return carry

    def fill_g_lo(u, carry):
        g_v[pl.ds(FILL_LO + u * 16, 16)] = c_lo
        return carry

    lax.fori_loop(0, FILL_HI // 16, fill_g_hi, 0)
    lax.fori_loop(0, (GPAD - FILL_LO) // 16, fill_g_lo, 0)
    g_v[pl.ds(FILL_HI, 16)] = c_hi
    for w0 in range(1920, FILL_LO, 16):
        g_v[pl.ds(w0, 16)] = lax.rev(col_v[pl.ds(2160 - w0, 16)], (0,))

    def bsrc_qg(qg, carry):
        for kt in range(3):
            for r8 in range(8):
                base_o = kt * 1024 + r8 * 128
                base_i = 1919 + 128 * kt - r8
                for m in range(8):
                    bsrc_v[pl.ds(qg * 3072 + base_o + m * 16, 16)] = (
                        g_v[pl.ds(base_i - 8 * qg + m * 16, 16)]
                    )
        return carry

    lax.fori_loop(0, 16, bsrc_qg, 0)

    def fill_c(u, carry):
        chi_v[pl.ds(u * 16, 16)] = c_hi
        clo_v[pl.ds(u * 16, 16)] = c_lo
        return carry

    lax.fori_loop(0, CRUN * 1024 // 16, fill_c, 0)

    def emit(cv):
        for b in range(NBLK // 2):
            i0 = cv * (Q // 2) + b * 128
            c0 = max(0, i0 - 128)
            wend = 256 if i0 == 0 else min(K, c0 + 384)
            kt0, nkt = c0 // 128, (wend - c0) // 128
            boff = 1024 if i0 == 0 else 0

            def qg_body(qg, carry, i0=i0, kt0=kt0, nkt=nkt, boff=boff):
                qga = i0 // 8 + qg
                dbase = (s * (Q // 8) + qga) * NKT * 1024
                descs = []
                for k0 in range(0, kt0, CRUN):
                    w = min(CRUN, kt0 - k0) * 1024
                    descs.append(pltpu.async_copy(
                        chi_v.at[pl.ds(0, w)],
                        out_hbm.at[pl.ds(dbase + k0 * 1024, w)], sem))
                descs.append(pltpu.async_copy(
                    bsrc_v.at[pl.ds(qg * 3072 + boff, nkt * 1024)],
                    out_hbm.at[pl.ds(dbase + kt0 * 1024, nkt * 1024)], sem))
                for k0 in range(kt0 + nkt, NKT, CRUN):
                    w = min(CRUN, NKT - k0) * 1024
                    descs.append(pltpu.async_copy(
                        clo_v.at[pl.ds(0, w)],
                        out_hbm.at[pl.ds(dbase + k0 * 1024, w)], sem))
                for dcp in descs:
                    dcp.wait()
                return carry

            lax.fori_loop(0, 16, qg_body, 0)

    for cv in range(2):
        @pl.when(c == cv)
        def _go(cv=cv):
            emit(cv)


@jax.jit
def _launch(bias):
    bias_t = jnp.pad(bias.T, ((0, 0), (0, TPAD - T)))
    fn = pl.kernel(
        _rpb_sc,
        mesh=plsc.VectorSubcoreMesh(core_axis_name="c", subcore_axis_name="s"),
        out_type=jax.ShapeDtypeStruct((H * Q * K,), jnp.float32),
        scratch_types=[
            pltpu.VMEM((TPAD,), jnp.float32),
            pltpu.VMEM((GPAD,), jnp.float32),
            pltpu.VMEM((16 * 3072,), jnp.float32),
            pltpu.VMEM((CRUN * 1024,), jnp.float32),
            pltpu.VMEM((CRUN * 1024,), jnp.float32),
            pltpu.SemaphoreType.DMA,
        ],
    )
    flat = fn(bias_t)
    return (flat.reshape(H, Q // 8, NKT, 8, 128)
            .transpose(0, 1, 3, 2, 4)
            .reshape(H, Q, K))


def kernel(q_len, k_len, bias):
    return _launch(bias)


# speedup vs baseline: 3.7941x; 1.0592x over previous
import jax
import jax.numpy as jnp
from jax import lax
from jax.experimental import pallas as pl
from jax.experimental.pallas import tpu as pltpu
from jax.experimental.pallas import tpu_sc as plsc

H = 16
Q = 2048
K = 2048
T = 257
D = (T - 1) // 2
TPAD = 264
GPAD = 4128
M = K - 1
FILL_HI = 1904
FILL_LO = 2176
NKT = K // 128
CRUN = 14
NBLK = Q // 128


def _block_geom(i0):
    c0 = max(0, i0 - 128)
    wend = 256 if i0 == 0 else min(K, c0 + 384)
    kt0, nkt = c0 // 128, (wend - c0) // 128
    boff = 1024 if i0 == 0 else 0
    return kt0, nkt, boff, kt0, NKT - kt0 - nkt


def _bcast_lane(v, lane):
    idx = jnp.full((16, 1), lane, jnp.int32)
    dnums = lax.GatherDimensionNumbers(
        offset_dims=(), collapsed_slice_dims=(0,), start_index_map=(0,)
    )
    return lax.gather(v, idx, dnums, slice_sizes=(1,),
                      mode=lax.GatherScatterMode.PROMISE_IN_BOUNDS)


def _rpb_sc(bias_hbm, out_hbm, col_v, g_v, bsrc_v, chi_v, clo_v, sem):
    c = lax.axis_index("c")
    s = lax.axis_index("s")
    pltpu.sync_copy(bias_hbm.at[s], col_v)

    c_hi = _bcast_lane(col_v[pl.ds(248, 16)], 8)
    c_lo = _bcast_lane(col_v[pl.ds(0, 16)], 0)

    def fill_c(u, carry):
        chi_v[pl.ds(u * 16, 16)] = c_hi
        clo_v[pl.ds(u * 16, 16)] = c_lo
        return carry

    lax.fori_loop(0, CRUN * 1024 // 16, fill_c, 0)

    def const_descs(cv, b, issue):
        i0 = cv * (Q // 2) + b * 128
        kt0, nkt, _, w_l, w_r = _block_geom(i0)

        def qg_body(qg, carry):
            dbase = (s * (Q // 8) + i0 // 8 + qg) * NKT * 1024
            mk = pltpu.async_copy if issue else pltpu.make_async_copy
            if w_l:
                d = mk(chi_v.at[pl.ds(0, w_l * 1024)],
                       out_hbm.at[pl.ds(dbase, w_l * 1024)], sem)
                if not issue:
                    d.wait()
            if w_r:
                d = mk(clo_v.at[pl.ds(0, w_r * 1024)],
                       out_hbm.at[pl.ds(dbase + (kt0 + nkt) * 1024, w_r * 1024)],
                       sem)
                if not issue:
                    d.wait()
            return carry

        lax.fori_loop(0, 16, qg_body, 0)

    for cv in range(2):
        @pl.when(c == cv)
        def _a(cv=cv):
            for b in range(NBLK // 2):
                const_descs(cv, b, issue=True)
                if b > 0:
                    const_descs(cv, b - 1, issue=False)

    def fill_g_hi(u, carry):
        g_v[pl.ds(u * 16, 16)] = c_hi
        return carry

    def fill_g_lo(u, carry):
        g_v[pl.ds(FILL_LO + u * 16, 16)] = c_lo
        return carry

    lax.fori_loop(0, FILL_HI // 16, fill_g_hi, 0)
    lax.fori_loop(0, (GPAD - FILL_LO) // 16, fill_g_lo, 0)
    g_v[pl.ds(FILL_HI, 16)] = c_hi
    for w0 in range(1920, FILL_LO, 16):
        g_v[pl.ds(w0, 16)] = lax.rev(col_v[pl.ds(2160 - w0, 16)], (0,))

    def bsrc_qg(qg, carry):
        for kt in range(3):
            for r8 in range(8):
                base_o = kt * 1024 + r8 * 128
                base_i = 1919 + 128 * kt - r8
                for m in range(8):
                    bsrc_v[pl.ds(qg * 3072 + base_o + m * 16, 16)] = (
                        g_v[pl.ds(base_i - 8 * qg + m * 16, 16)]
                    )
        return carry

    lax.fori_loop(0, 16, bsrc_qg, 0)

    for cv in range(2):
        @pl.when(c == cv)
        def _c(cv=cv):
            for b in range(NBLK // 2):
                i0 = cv * (Q // 2) + b * 128
                kt0, nkt, boff, _, _ = _block_geom(i0)

                def qg_body(qg, carry, i0=i0, kt0=kt0, nkt=nkt, boff=boff):
                    dbase = (s * (Q // 8) + i0 // 8 + qg) * NKT * 1024
                    pltpu.async_copy(
                        bsrc_v.at[pl.ds(qg * 3072 + boff, nkt * 1024)],
                        out_hbm.at[pl.ds(dbase + kt0 * 1024, nkt * 1024)],
                        sem).wait()
                    return carry

                lax.fori_loop(0, 16, qg_body, 0)
            const_descs(cv, NBLK // 2 - 1, issue=False)


@jax.jit
def _launch(bias):
    bias_t = jnp.pad(bias.T, ((0, 0), (0, TPAD - T)))
    fn = pl.kernel(
        _rpb_sc,
        mesh=plsc.VectorSubcoreMesh(core_axis_name="c", subcore_axis_name="s"),
        out_type=jax.ShapeDtypeStruct((H * Q * K,), jnp.float32),
        scratch_types=[
            pltpu.VMEM((TPAD,), jnp.float32),
            pltpu.VMEM((GPAD,), jnp.float32),
            pltpu.VMEM((16 * 3072,), jnp.float32),
            pltpu.VMEM((CRUN * 1024,), jnp.float32),
            pltpu.VMEM((CRUN * 1024,), jnp.float32),
            pltpu.SemaphoreType.DMA,
        ],
    )
    flat = fn(bias_t)
    return (flat.reshape(H, Q // 8, NKT, 8, 128)
            .transpose(0, 1, 3, 2, 4)
            .reshape(H, Q, K))


def kernel(q_len, k_len, bias):
    return _launch(bias)
